# baseline (speedup 1.0000x reference)
import functools

import jax
import jax.numpy as jnp
from jax import lax
from jax.experimental import pallas as pl
from jax.experimental.pallas import tpu as pltpu
from jax.experimental.pallas import tpu_sc as plsc


def _stats_kernel(x_ref, m_ref, mem_ref, pooled_ref, praw_ref,
                  sxa, csa, dna, *, nb, d):
    p = pl.program_id(0)
    np_ = pl.num_programs(0)
    spb = np_ // nb
    b = p // spb
    xv = x_ref[...]
    mv = m_ref[...]
    scores = jax.lax.dot_general(
        xv, mem_ref[...], (((1,), (1,)), ((), ())),
        preferred_element_type=jnp.float32) * (1.0 / (d ** 0.5))
    mx = jnp.max(scores, axis=1, keepdims=True)
    e = jnp.exp(scores - mx)
    attn = e / jnp.sum(e, axis=1, keepdims=True)
    sx = jnp.sum(xv * mv, axis=0)
    cs = jnp.sum(attn * mv, axis=0)
    dn = jnp.sum(mv)
    bmask = (jax.lax.broadcasted_iota(jnp.int32, (nb, 1), 0) == b
             ).astype(jnp.float32)

    @pl.when(p == 0)
    def _():
        sxa[...] = jnp.zeros_like(sxa)
        csa[...] = jnp.zeros_like(csa)
        dna[...] = jnp.zeros_like(dna)

    sxa[...] += bmask * sx[None, :]
    csa[...] += bmask * cs[None, :]
    dna[...] += bmask * dn

    @pl.when(p == np_ - 1)
    def _():
        dnc = jnp.maximum(dna[...], 1e-6)
        praw = sxa[...] / dnc
        memsum = jnp.dot(csa[...], mem_ref[...],
                         preferred_element_type=jnp.float32)
        pooled_ref[...] = praw + 0.2 * (memsum / dnc)
        praw_ref[...] = praw


def _router_sc_kernel(praw_hbm, pooled_hbm, wrt_hbm, brv_hbm, wopt_hbm,
                      wtt_hbm, opemb_hbm, taskemb_hbm, ba1_hbm, bl1_hbm,
                      ba2_hbm, bl2_hbm,
                      biasA_hbm, biasL_hbm, bias2_hbm, w_hbm,
                      pr_v, po_v, wrt_v, brv_v, wopt_v, wtt_v, opemb_v,
                      taskemb_v, ba1_v, bl1_v, ba2_v, bl2_v,
                      bA_v, bL_v, b2_v, w_v, sem, *, nb, d, df, nc):
    c = lax.axis_index("c")
    s = lax.axis_index("s")
    wid = s * nc + c

    @pl.when(wid < nb)
    def _():
        b = wid
        copies = [
            pltpu.async_copy(praw_hbm.at[pl.ds(b, 1)], pr_v, sem),
            pltpu.async_copy(pooled_hbm.at[pl.ds(b, 1)], po_v, sem),
            pltpu.async_copy(wrt_hbm, wrt_v, sem),
            pltpu.async_copy(brv_hbm, brv_v, sem),
            pltpu.async_copy(wopt_hbm, wopt_v, sem),
            pltpu.async_copy(wtt_hbm, wtt_v, sem),
            pltpu.async_copy(opemb_hbm, opemb_v, sem),
            pltpu.async_copy(taskemb_hbm, taskemb_v, sem),
            pltpu.async_copy(ba1_hbm, ba1_v, sem),
            pltpu.async_copy(bl1_hbm, bl1_v, sem),
            pltpu.async_copy(ba2_hbm, ba2_v, sem),
            pltpu.async_copy(bl2_hbm, bl2_v, sem),
        ]
        for cp in copies:
            cp.wait()

        zeros = jnp.zeros((16,), jnp.float32)

        def dot_body(i, accs):
            sl = pl.ds(i * 16, 16)
            pc = po_v[0, sl]
            prc = pr_v[0, sl]
            r0, r1, o0, o1, o2, o3, t0, t1, t2, t3 = accs
            r0 = r0 + pc * wrt_v[0, sl]
            r1 = r1 + pc * wrt_v[1, sl]
            o0 = o0 + prc * wopt_v[0, sl]
            o1 = o1 + prc * wopt_v[1, sl]
            o2 = o2 + prc * wopt_v[2, sl]
            o3 = o3 + prc * wopt_v[3, sl]
            t0 = t0 + prc * wtt_v[0, sl]
            t1 = t1 + prc * wtt_v[1, sl]
            t2 = t2 + prc * wtt_v[2, sl]
            t3 = t3 + prc * wtt_v[3, sl]
            return (r0, r1, o0, o1, o2, o3, t0, t1, t2, t3)

        accs = lax.fori_loop(0, d // 16, dot_body, (zeros,) * 10)
        iota16 = lax.broadcasted_iota(jnp.int32, (16,), 0)
        neg = jnp.float32(-1e30)

        brvec = brv_v[0, :]
        l0 = jnp.sum(accs[0]) + brvec[0]
        l1 = jnp.sum(accs[1]) + brvec[1]
        lv = jnp.where(iota16 == 0, l0, jnp.where(iota16 == 1, l1, neg))
        ev = jnp.exp(lv - jnp.max(lv))
        wv = ev / jnp.sum(ev)
        w_v[0, :] = wv
        w0 = wv[0]
        w1 = wv[1]

        def amax4(a0, a1, a2, a3):
            v = jnp.where(
                iota16 == 0, a0,
                jnp.where(iota16 == 1, a1,
                          jnp.where(iota16 == 2, a2,
                                    jnp.where(iota16 == 3, a3, neg))))
            cand = jnp.where(v == jnp.max(v), iota16, 16)
            return jnp.min(cand)

        op = amax4(jnp.sum(accs[2]), jnp.sum(accs[3]),
                   jnp.sum(accs[4]), jnp.sum(accs[5]))
        task = amax4(jnp.sum(accs[6]), jnp.sum(accs[7]),
                     jnp.sum(accs[8]), jnp.sum(accs[9]))
        oh = [jnp.where(op == i, 1.0, 0.0).astype(jnp.float32)
              for i in range(4)]
        th = [jnp.where(task == i, 1.0, 0.0).astype(jnp.float32)
              for i in range(4)]

        def bias_body(i, _):
            sl = pl.ds(i * 16, 16)
            bA_v[0, sl] = (ba1_v[0, sl]
                           + oh[0] * opemb_v[0, sl] + oh[1] * opemb_v[1, sl]
                           + oh[2] * opemb_v[2, sl] + oh[3] * opemb_v[3, sl])
            bL_v[0, sl] = (bl1_v[0, sl]
                           + th[0] * taskemb_v[0, sl] + th[1] * taskemb_v[1, sl]
                           + th[2] * taskemb_v[2, sl] + th[3] * taskemb_v[3, sl])
            return 0

        lax.fori_loop(0, df // 16, bias_body, 0)

        def b2_body(i, _):
            sl = pl.ds(i * 16, 16)
            b2_v[0, sl] = w0 * ba2_v[0, sl] + w1 * bl2_v[0, sl]
            return 0

        lax.fori_loop(0, d // 16, b2_body, 0)

        pltpu.sync_copy(bA_v, biasA_hbm.at[pl.ds(b, 1)])
        pltpu.sync_copy(bL_v, biasL_hbm.at[pl.ds(b, 1)])
        pltpu.sync_copy(b2_v, bias2_hbm.at[pl.ds(b, 1)])
        pltpu.sync_copy(w_v, w_hbm.at[pl.ds(b, 1)])


def _mlp_kernel(x_ref, wa1_ref, wl1_ref, wa2_ref, wl2_ref,
                bA_ref, bL_ref, b2_ref, w_ref,
                wf_ref, bf_ref, wc1_ref, wc2_ref, o_ref, *, tpb, kd):
    t = pl.program_id(0)
    b = t // tpb
    xv = x_ref[...].astype(jnp.bfloat16)
    df = bA_ref.shape[-1]
    w0 = w_ref[b, 0].astype(jnp.bfloat16)
    w1 = w_ref[b, 1].astype(jnp.bfloat16)
    fused = b2_ref[0, 0, :][None, :] * jnp.ones_like(x_ref[..., :1])
    for kk in range(df // kd):
        sl = slice(kk * kd, (kk + 1) * kd)
        ha = jnp.dot(xv, wa1_ref[:, sl],
                     preferred_element_type=jnp.float32).astype(jnp.bfloat16)
        ha = jnp.maximum(ha + bA_ref[0, 0, sl].astype(jnp.bfloat16)[None, :],
                         jnp.bfloat16(0.0)) * w0
        hl = jnp.dot(xv, wl1_ref[:, sl],
                     preferred_element_type=jnp.float32).astype(jnp.bfloat16)
        hl = jax.nn.gelu(hl + bL_ref[0, 0, sl].astype(jnp.bfloat16)[None, :],
                         approximate=True) * w1
        fused = fused + jnp.dot(ha, wa2_ref[sl, :],
                                preferred_element_type=jnp.float32)
        fused = fused + jnp.dot(hl, wl2_ref[sl, :],
                                preferred_element_type=jnp.float32)
    y = jnp.dot(fused.astype(jnp.bfloat16), wf_ref[...],
                preferred_element_type=jnp.float32) + bf_ref[...]
    th = jnp.tanh(jnp.dot(y.astype(jnp.bfloat16), wc1_ref[...],
                          preferred_element_type=jnp.float32))
    ctx = jnp.dot(th.astype(jnp.bfloat16), wc2_ref[...],
                  preferred_element_type=jnp.float32)
    o_ref[...] = (y + ctx) * 0.5


def kernel(hidden_states, attention_mask, memory, Wr, br, Wop, Wtask, OpEmb,
           TaskEmb, Wa1, ba1, Wa2, ba2, Wl1, bl1, Wl2, bl2, Wf, bf, Wc1, Wc2,
           interpret=False):
    B, S, D = hidden_states.shape
    M = memory.shape[0]
    DF = Wa1.shape[1]
    T = B * S
    TS = 512
    TT = 512
    tpb = S // TT

    x = hidden_states.reshape(T, D)
    mask2 = attention_mask.reshape(T, 1)

    pooled, praw = pl.pallas_call(
        functools.partial(_stats_kernel, nb=B, d=D),
        grid=(T // TS,),
        in_specs=[
            pl.BlockSpec((TS, D), lambda t: (t, 0)),
            pl.BlockSpec((TS, 1), lambda t: (t, 0)),
            pl.BlockSpec((M, D), lambda t: (0, 0)),
        ],
        out_specs=[
            pl.BlockSpec((B, D), lambda t: (0, 0)),
            pl.BlockSpec((B, D), lambda t: (0, 0)),
        ],
        out_shape=[
            jax.ShapeDtypeStruct((B, D), jnp.float32),
            jax.ShapeDtypeStruct((B, D), jnp.float32),
        ],
        scratch_shapes=[
            pltpu.VMEM((B, D), jnp.float32),
            pltpu.VMEM((B, M), jnp.float32),
            pltpu.VMEM((B, 1), jnp.float32),
        ],
        compiler_params=pltpu.CompilerParams(
            dimension_semantics=("arbitrary",)),
        interpret=interpret,
    )(x, mask2, memory)

    info = plsc.get_sparse_core_info()
    nc = info.num_cores
    mesh = plsc.VectorSubcoreMesh(core_axis_name="c", subcore_axis_name="s")
    router = functools.partial(
        pl.kernel,
        mesh=mesh,
        compiler_params=pltpu.CompilerParams(needs_layout_passes=False),
        out_type=[
            jax.ShapeDtypeStruct((B, DF), jnp.float32),
            jax.ShapeDtypeStruct((B, DF), jnp.float32),
            jax.ShapeDtypeStruct((B, D), jnp.float32),
            jax.ShapeDtypeStruct((B, 16), jnp.float32),
        ],
        scratch_types=[
            pltpu.VMEM((1, D), jnp.float32),
            pltpu.VMEM((1, D), jnp.float32),
            pltpu.VMEM((2, D), jnp.float32),
            pltpu.VMEM((1, 16), jnp.float32),
            pltpu.VMEM((4, D), jnp.float32),
            pltpu.VMEM((4, D), jnp.float32),
            pltpu.VMEM((4, DF), jnp.float32),
            pltpu.VMEM((4, DF), jnp.float32),
            pltpu.VMEM((1, DF), jnp.float32),
            pltpu.VMEM((1, DF), jnp.float32),
            pltpu.VMEM((1, D), jnp.float32),
            pltpu.VMEM((1, D), jnp.float32),
            pltpu.VMEM((1, DF), jnp.float32),
            pltpu.VMEM((1, DF), jnp.float32),
            pltpu.VMEM((1, D), jnp.float32),
            pltpu.VMEM((1, 16), jnp.float32),
            pltpu.SemaphoreType.DMA,
        ],
    )(functools.partial(_router_sc_kernel, nb=B, d=D, df=DF, nc=nc))
    brv = jnp.zeros((1, 16), jnp.float32).at[0, :2].set(br)
    biasA, biasL, bias2, w16 = router(
        praw, pooled, Wr.T, brv, Wop.T, Wtask.T, OpEmb, TaskEmb,
        ba1.reshape(1, DF), bl1.reshape(1, DF), ba2.reshape(1, D),
        bl2.reshape(1, D))

    out = pl.pallas_call(
        functools.partial(_mlp_kernel, tpb=tpb, kd=2048),
        grid=(T // TT,),
        in_specs=[
            pl.BlockSpec((TT, D), lambda t: (t, 0)),
            pl.BlockSpec((D, DF), lambda t: (0, 0)),
            pl.BlockSpec((D, DF), lambda t: (0, 0)),
            pl.BlockSpec((DF, D), lambda t: (0, 0)),
            pl.BlockSpec((DF, D), lambda t: (0, 0)),
            pl.BlockSpec((1, 1, DF), lambda t, _tpb=tpb: (t // _tpb, 0, 0)),
            pl.BlockSpec((1, 1, DF), lambda t, _tpb=tpb: (t // _tpb, 0, 0)),
            pl.BlockSpec((1, 1, D), lambda t, _tpb=tpb: (t // _tpb, 0, 0)),
            pl.BlockSpec(memory_space=pltpu.SMEM),
            pl.BlockSpec((D, D), lambda t: (0, 0)),
            pl.BlockSpec((1, D), lambda t: (0, 0)),
            pl.BlockSpec((D, D), lambda t: (0, 0)),
            pl.BlockSpec((D, D), lambda t: (0, 0)),
        ],
        out_specs=pl.BlockSpec((TT, D), lambda t: (t, 0)),
        out_shape=jax.ShapeDtypeStruct((T, D), jnp.float32),
        compiler_params=pltpu.CompilerParams(
            dimension_semantics=("arbitrary",)),
        interpret=interpret,
    )(x, Wa1.astype(jnp.bfloat16), Wl1.astype(jnp.bfloat16),
      Wa2.astype(jnp.bfloat16), Wl2.astype(jnp.bfloat16),
      biasA.reshape(B, 1, DF), biasL.reshape(B, 1, DF),
      bias2.reshape(B, 1, D), w16,
      Wf.astype(jnp.bfloat16), bf.reshape(1, D),
      Wc1.astype(jnp.bfloat16), Wc2.astype(jnp.bfloat16))

    return out.reshape(B, S, D)


# speedup vs baseline: 1.0005x; 1.0005x over previous
import functools

import jax
import jax.numpy as jnp
from jax import lax
from jax.experimental import pallas as pl
from jax.experimental.pallas import tpu as pltpu
from jax.experimental.pallas import tpu_sc as plsc


def _stats_kernel(x_ref, m_ref, mem_ref, pooled_ref, praw_ref,
                  sxa, csa, dna, *, nb, d):
    p = pl.program_id(0)
    np_ = pl.num_programs(0)
    spb = np_ // nb
    b = p // spb
    xv = x_ref[...]
    mv = m_ref[...]
    scores = jax.lax.dot_general(
        xv, mem_ref[...], (((1,), (1,)), ((), ())),
        preferred_element_type=jnp.float32) * (1.0 / (d ** 0.5))
    mx = jnp.max(scores, axis=1, keepdims=True)
    e = jnp.exp(scores - mx)
    attn = e / jnp.sum(e, axis=1, keepdims=True)
    sx = jnp.sum(xv * mv, axis=0)
    cs = jnp.sum(attn * mv, axis=0)
    dn = jnp.sum(mv)
    bmask = (jax.lax.broadcasted_iota(jnp.int32, (nb, 1), 0) == b
             ).astype(jnp.float32)

    @pl.when(p == 0)
    def _():
        sxa[...] = jnp.zeros_like(sxa)
        csa[...] = jnp.zeros_like(csa)
        dna[...] = jnp.zeros_like(dna)

    sxa[...] += bmask * sx[None, :]
    csa[...] += bmask * cs[None, :]
    dna[...] += bmask * dn

    @pl.when(p == np_ - 1)
    def _():
        dnc = jnp.maximum(dna[...], 1e-6)
        praw = sxa[...] / dnc
        memsum = jnp.dot(csa[...], mem_ref[...],
                         preferred_element_type=jnp.float32)
        pooled_ref[...] = praw + 0.2 * (memsum / dnc)
        praw_ref[...] = praw


def _router_sc_kernel(praw_hbm, pooled_hbm, wrt_hbm, brv_hbm, wopt_hbm,
                      wtt_hbm, opemb_hbm, taskemb_hbm, ba1_hbm, bl1_hbm,
                      ba2_hbm, bl2_hbm,
                      biasA_hbm, biasL_hbm, bias2_hbm, w_hbm,
                      pr_v, po_v, wrt_v, brv_v, wopt_v, wtt_v, opemb_v,
                      taskemb_v, ba1_v, bl1_v, ba2_v, bl2_v,
                      bA_v, bL_v, b2_v, w_v, sem, *, nb, d, df, nc):
    c = lax.axis_index("c")
    s = lax.axis_index("s")
    wid = s * nc + c

    @pl.when(wid < nb)
    def _():
        b = wid
        copies = [
            pltpu.async_copy(praw_hbm.at[pl.ds(b, 1)], pr_v, sem),
            pltpu.async_copy(pooled_hbm.at[pl.ds(b, 1)], po_v, sem),
            pltpu.async_copy(wrt_hbm, wrt_v, sem),
            pltpu.async_copy(brv_hbm, brv_v, sem),
            pltpu.async_copy(wopt_hbm, wopt_v, sem),
            pltpu.async_copy(wtt_hbm, wtt_v, sem),
            pltpu.async_copy(opemb_hbm, opemb_v, sem),
            pltpu.async_copy(taskemb_hbm, taskemb_v, sem),
            pltpu.async_copy(ba1_hbm, ba1_v, sem),
            pltpu.async_copy(bl1_hbm, bl1_v, sem),
            pltpu.async_copy(ba2_hbm, ba2_v, sem),
            pltpu.async_copy(bl2_hbm, bl2_v, sem),
        ]
        for cp in copies:
            cp.wait()

        zeros = jnp.zeros((16,), jnp.float32)

        def dot_body(i, accs):
            sl = pl.ds(i * 16, 16)
            pc = po_v[0, sl]
            prc = pr_v[0, sl]
            r0, r1, o0, o1, o2, o3, t0, t1, t2, t3 = accs
            r0 = r0 + pc * wrt_v[0, sl]
            r1 = r1 + pc * wrt_v[1, sl]
            o0 = o0 + prc * wopt_v[0, sl]
            o1 = o1 + prc * wopt_v[1, sl]
            o2 = o2 + prc * wopt_v[2, sl]
            o3 = o3 + prc * wopt_v[3, sl]
            t0 = t0 + prc * wtt_v[0, sl]
            t1 = t1 + prc * wtt_v[1, sl]
            t2 = t2 + prc * wtt_v[2, sl]
            t3 = t3 + prc * wtt_v[3, sl]
            return (r0, r1, o0, o1, o2, o3, t0, t1, t2, t3)

        accs = lax.fori_loop(0, d // 16, dot_body, (zeros,) * 10)
        iota16 = lax.broadcasted_iota(jnp.int32, (16,), 0)
        neg = jnp.float32(-1e30)

        brvec = brv_v[0, :]
        l0 = jnp.sum(accs[0]) + brvec[0]
        l1 = jnp.sum(accs[1]) + brvec[1]
        lv = jnp.where(iota16 == 0, l0, jnp.where(iota16 == 1, l1, neg))
        ev = jnp.exp(lv - jnp.max(lv))
        wv = ev / jnp.sum(ev)
        w_v[0, :] = wv
        w0 = wv[0]
        w1 = wv[1]

        def amax4(a0, a1, a2, a3):
            v = jnp.where(
                iota16 == 0, a0,
                jnp.where(iota16 == 1, a1,
                          jnp.where(iota16 == 2, a2,
                                    jnp.where(iota16 == 3, a3, neg))))
            cand = jnp.where(v == jnp.max(v), iota16, 16)
            return jnp.min(cand)

        op = amax4(jnp.sum(accs[2]), jnp.sum(accs[3]),
                   jnp.sum(accs[4]), jnp.sum(accs[5]))
        task = amax4(jnp.sum(accs[6]), jnp.sum(accs[7]),
                     jnp.sum(accs[8]), jnp.sum(accs[9]))
        oh = [jnp.where(op == i, 1.0, 0.0).astype(jnp.float32)
              for i in range(4)]
        th = [jnp.where(task == i, 1.0, 0.0).astype(jnp.float32)
              for i in range(4)]

        def bias_body(i, _):
            sl = pl.ds(i * 16, 16)
            bA_v[0, sl] = (ba1_v[0, sl]
                           + oh[0] * opemb_v[0, sl] + oh[1] * opemb_v[1, sl]
                           + oh[2] * opemb_v[2, sl] + oh[3] * opemb_v[3, sl])
            bL_v[0, sl] = (bl1_v[0, sl]
                           + th[0] * taskemb_v[0, sl] + th[1] * taskemb_v[1, sl]
                           + th[2] * taskemb_v[2, sl] + th[3] * taskemb_v[3, sl])
            return 0

        lax.fori_loop(0, df // 16, bias_body, 0)

        def b2_body(i, _):
            sl = pl.ds(i * 16, 16)
            b2_v[0, sl] = w0 * ba2_v[0, sl] + w1 * bl2_v[0, sl]
            return 0

        lax.fori_loop(0, d // 16, b2_body, 0)

        pltpu.sync_copy(bA_v, biasA_hbm.at[pl.ds(b, 1)])
        pltpu.sync_copy(bL_v, biasL_hbm.at[pl.ds(b, 1)])
        pltpu.sync_copy(b2_v, bias2_hbm.at[pl.ds(b, 1)])
        pltpu.sync_copy(w_v, w_hbm.at[pl.ds(b, 1)])


def _mlp_kernel(x_ref, wa1_ref, wl1_ref, wa2_ref, wl2_ref,
                bA_ref, bL_ref, b2_ref, w_ref,
                wf_ref, bf_ref, wc1_ref, wc2_ref, o_ref, *, tpb, kd):
    t = pl.program_id(0)
    b = t // tpb
    xv = x_ref[...].astype(jnp.bfloat16)
    df = bA_ref.shape[-1]
    w0 = w_ref[b, 0].astype(jnp.bfloat16)
    w1 = w_ref[b, 1].astype(jnp.bfloat16)
    fused = b2_ref[0, 0, :][None, :] * jnp.ones_like(x_ref[..., :1])
    for kk in range(df // kd):
        sl = slice(kk * kd, (kk + 1) * kd)
        ha = jnp.dot(xv, wa1_ref[:, sl],
                     preferred_element_type=jnp.float32).astype(jnp.bfloat16)
        ha = jnp.maximum(ha + bA_ref[0, 0, sl].astype(jnp.bfloat16)[None, :],
                         jnp.bfloat16(0.0)) * w0
        hl = jnp.dot(xv, wl1_ref[:, sl],
                     preferred_element_type=jnp.float32).astype(jnp.bfloat16)
        hl = jax.nn.gelu(hl + bL_ref[0, 0, sl].astype(jnp.bfloat16)[None, :],
                         approximate=True) * w1
        fused = fused + jnp.dot(ha, wa2_ref[sl, :],
                                preferred_element_type=jnp.float32)
        fused = fused + jnp.dot(hl, wl2_ref[sl, :],
                                preferred_element_type=jnp.float32)
    y = jnp.dot(fused.astype(jnp.bfloat16), wf_ref[...],
                preferred_element_type=jnp.float32) + bf_ref[...]
    th = jnp.tanh(jnp.dot(y.astype(jnp.bfloat16), wc1_ref[...],
                          preferred_element_type=jnp.float32))
    ctx = jnp.dot(th.astype(jnp.bfloat16), wc2_ref[...],
                  preferred_element_type=jnp.float32)
    o_ref[...] = (y + ctx) * 0.5


def kernel(hidden_states, attention_mask, memory, Wr, br, Wop, Wtask, OpEmb,
           TaskEmb, Wa1, ba1, Wa2, ba2, Wl1, bl1, Wl2, bl2, Wf, bf, Wc1, Wc2,
           interpret=False):
    B, S, D = hidden_states.shape
    M = memory.shape[0]
    DF = Wa1.shape[1]
    T = B * S
    TS = 512
    TT = 512
    tpb = S // TT

    x = hidden_states.reshape(T, D)
    mask2 = attention_mask.reshape(T, 1)

    pooled, praw = pl.pallas_call(
        functools.partial(_stats_kernel, nb=B, d=D),
        grid=(T // TS,),
        in_specs=[
            pl.BlockSpec((TS, D), lambda t: (t, 0)),
            pl.BlockSpec((TS, 1), lambda t: (t, 0)),
            pl.BlockSpec((M, D), lambda t: (0, 0)),
        ],
        out_specs=[
            pl.BlockSpec((B, D), lambda t: (0, 0)),
            pl.BlockSpec((B, D), lambda t: (0, 0)),
        ],
        out_shape=[
            jax.ShapeDtypeStruct((B, D), jnp.float32),
            jax.ShapeDtypeStruct((B, D), jnp.float32),
        ],
        scratch_shapes=[
            pltpu.VMEM((B, D), jnp.float32),
            pltpu.VMEM((B, M), jnp.float32),
            pltpu.VMEM((B, 1), jnp.float32),
        ],
        compiler_params=pltpu.CompilerParams(
            dimension_semantics=("arbitrary",)),
        interpret=interpret,
    )(x, mask2, memory)

    info = plsc.get_sparse_core_info()
    nc = info.num_cores
    mesh = plsc.VectorSubcoreMesh(core_axis_name="c", subcore_axis_name="s")
    router = functools.partial(
        pl.kernel,
        mesh=mesh,
        compiler_params=pltpu.CompilerParams(needs_layout_passes=False),
        out_type=[
            jax.ShapeDtypeStruct((B, DF), jnp.float32),
            jax.ShapeDtypeStruct((B, DF), jnp.float32),
            jax.ShapeDtypeStruct((B, D), jnp.float32),
            jax.ShapeDtypeStruct((B, 16), jnp.float32),
        ],
        scratch_types=[
            pltpu.VMEM((1, D), jnp.float32),
            pltpu.VMEM((1, D), jnp.float32),
            pltpu.VMEM((2, D), jnp.float32),
            pltpu.VMEM((1, 16), jnp.float32),
            pltpu.VMEM((4, D), jnp.float32),
            pltpu.VMEM((4, D), jnp.float32),
            pltpu.VMEM((4, DF), jnp.float32),
            pltpu.VMEM((4, DF), jnp.float32),
            pltpu.VMEM((1, DF), jnp.float32),
            pltpu.VMEM((1, DF), jnp.float32),
            pltpu.VMEM((1, D), jnp.float32),
            pltpu.VMEM((1, D), jnp.float32),
            pltpu.VMEM((1, DF), jnp.float32),
            pltpu.VMEM((1, DF), jnp.float32),
            pltpu.VMEM((1, D), jnp.float32),
            pltpu.VMEM((1, 16), jnp.float32),
            pltpu.SemaphoreType.DMA,
        ],
    )(functools.partial(_router_sc_kernel, nb=B, d=D, df=DF, nc=nc))
    brv = jnp.zeros((1, 16), jnp.float32).at[0, :2].set(br)
    biasA, biasL, bias2, w16 = router(
        praw, pooled, Wr.T, brv, Wop.T, Wtask.T, OpEmb, TaskEmb,
        ba1.reshape(1, DF), bl1.reshape(1, DF), ba2.reshape(1, D),
        bl2.reshape(1, D))

    Wa1, Wl1, Wa2, Wl2, Wf, Wc1, Wc2, _ = jax.lax.optimization_barrier(
        (Wa1, Wl1, Wa2, Wl2, Wf, Wc1, Wc2, praw))

    out = pl.pallas_call(
        functools.partial(_mlp_kernel, tpb=tpb, kd=2048),
        grid=(T // TT,),
        in_specs=[
            pl.BlockSpec((TT, D), lambda t: (t, 0)),
            pl.BlockSpec((D, DF), lambda t: (0, 0)),
            pl.BlockSpec((D, DF), lambda t: (0, 0)),
            pl.BlockSpec((DF, D), lambda t: (0, 0)),
            pl.BlockSpec((DF, D), lambda t: (0, 0)),
            pl.BlockSpec((1, 1, DF), lambda t, _tpb=tpb: (t // _tpb, 0, 0)),
            pl.BlockSpec((1, 1, DF), lambda t, _tpb=tpb: (t // _tpb, 0, 0)),
            pl.BlockSpec((1, 1, D), lambda t, _tpb=tpb: (t // _tpb, 0, 0)),
            pl.BlockSpec(memory_space=pltpu.SMEM),
            pl.BlockSpec((D, D), lambda t: (0, 0)),
            pl.BlockSpec((1, D), lambda t: (0, 0)),
            pl.BlockSpec((D, D), lambda t: (0, 0)),
            pl.BlockSpec((D, D), lambda t: (0, 0)),
        ],
        out_specs=pl.BlockSpec((TT, D), lambda t: (t, 0)),
        out_shape=jax.ShapeDtypeStruct((T, D), jnp.float32),
        compiler_params=pltpu.CompilerParams(
            dimension_semantics=("arbitrary",)),
        interpret=interpret,
    )(x, Wa1.astype(jnp.bfloat16), Wl1.astype(jnp.bfloat16),
      Wa2.astype(jnp.bfloat16), Wl2.astype(jnp.bfloat16),
      biasA.reshape(B, 1, DF), biasL.reshape(B, 1, DF),
      bias2.reshape(B, 1, D), w16,
      Wf.astype(jnp.bfloat16), bf.reshape(1, D),
      Wc1.astype(jnp.bfloat16), Wc2.astype(jnp.bfloat16))

    return out.reshape(B, S, D)
